# Initial kernel scaffold; baseline (speedup 1.0000x reference)
import jax
import jax.numpy as jnp
from jax.experimental import pallas as pl


def kernel(input_ids, word_table, pos_table, tt_table, gamma, beta):
    raise NotImplementedError("write your pallas kernel here")


# speedup vs baseline: 2.0607x; 2.0607x over previous
import functools

import jax
import jax.numpy as jnp
from jax import lax
from jax.experimental import pallas as pl
from jax.experimental.pallas import tpu as pltpu
from jax.experimental.pallas import tpu_sc as plsc

B, S = 4, 2048
D = 128
EPS = 1e-07

NC, NS = 2, 16
NW = NC * NS
NT = B * S
TPW = NT // NW
CHUNK = 128
NCH = TPW // CHUNK
NG = D // 16


_GDNUMS = lax.GatherDimensionNumbers(
    offset_dims=(), collapsed_slice_dims=(0,), start_index_map=(0,))


def _allsum(v):
    for k in (8, 4, 2, 1):
        idx = lax.iota(jnp.int32, 16) ^ k
        p = lax.gather(v, idx[:, None], _GDNUMS, (1,),
                       mode=lax.GatherScatterMode.PROMISE_IN_BOUNDS)
        v = v + p
    return v


def _rsqrt(v):
    i = lax.bitcast_convert_type(v, jnp.int32)
    i = 0x5F3759DF - lax.shift_right_logical(i, 1)
    y = lax.bitcast_convert_type(i, jnp.float32)
    for _ in range(3):
        y = y * (1.5 - 0.5 * v * y * y)
    return y


def _body(ids_hbm, wt_hbm, pos_hbm, tt_hbm, g_hbm, b_hbm, out_hbm,
          idx_v, rows_v, pos_v, tt_v, g_v, b_v, sem):
    c = lax.axis_index("c")
    s = lax.axis_index("s")
    wid = s * NC + c
    base = wid * TPW
    pbase = lax.rem(wid, S // TPW) * TPW

    pltpu.sync_copy(ids_hbm.at[wid], idx_v)
    copies = [
        pltpu.async_copy(wt_hbm.at[idx_v.at[j]],
                         rows_v.at[pl.ds(j * CHUNK, CHUNK)], sem)
        for j in range(NCH)
    ]
    pltpu.sync_copy(pos_hbm.at[pl.ds(pbase, TPW)], pos_v)
    pltpu.sync_copy(tt_hbm.at[pl.ds(0, 1)], tt_v)
    pltpu.sync_copy(g_hbm, g_v)
    pltpu.sync_copy(b_hbm, b_v)
    for cp in copies:
        cp.wait()

    tt_g = [tt_v[0, pl.ds(16 * g, 16)] for g in range(NG)]
    ga_g = [g_v[pl.ds(16 * g, 16)] for g in range(NG)]
    be_g = [b_v[pl.ds(16 * g, 16)] for g in range(NG)]
    inv_d = 1.0 / D

    def row_fn(r, carry):
        xs = []
        for g in range(NG):
            x = rows_v[r, pl.ds(16 * g, 16)] + pos_v[r, pl.ds(16 * g, 16)]
            xs.append(x + tt_g[g])
        s1 = ((xs[0] + xs[1]) + (xs[2] + xs[3])) + \
             ((xs[4] + xs[5]) + (xs[6] + xs[7]))
        qs = [x * x for x in xs]
        s2 = ((qs[0] + qs[1]) + (qs[2] + qs[3])) + \
             ((qs[4] + qs[5]) + (qs[6] + qs[7]))
        mean = _allsum(s1) * inv_d
        ex2 = _allsum(s2) * inv_d
        y = _rsqrt(ex2 - mean * mean + EPS)
        for g in range(NG):
            rows_v[r, pl.ds(16 * g, 16)] = (xs[g] - mean) * y * ga_g[g] + be_g[g]
        return carry

    lax.fori_loop(0, TPW, row_fn, 0)
    pltpu.sync_copy(rows_v, out_hbm.at[pl.ds(base, TPW)])


@functools.partial(jax.jit, static_argnames=())
def kernel(input_ids, word_table, pos_table, tt_table, gamma, beta):
    ids = input_ids.astype(jnp.int32).reshape(NW, NCH, CHUNK)
    run = functools.partial(
        pl.kernel,
        out_type=jax.ShapeDtypeStruct((NT, D), jnp.float32),
        mesh=plsc.VectorSubcoreMesh(core_axis_name="c", subcore_axis_name="s"),
        scratch_types=[
            pltpu.VMEM((NCH, CHUNK), jnp.int32),
            pltpu.VMEM((TPW, D), jnp.float32),
            pltpu.VMEM((TPW, D), jnp.float32),
            pltpu.VMEM((1, D), jnp.float32),
            pltpu.VMEM((D,), jnp.float32),
            pltpu.VMEM((D,), jnp.float32),
            pltpu.SemaphoreType.DMA,
        ],
    )(_body)
    out = run(ids, word_table, pos_table, tt_table, gamma, beta)
    return out.reshape(B, S, D)
